# baseline (speedup 1.0000x reference)
import functools
import jax
import jax.numpy as jnp
from jax import lax
from jax.experimental import pallas as pl
from jax.experimental.pallas import tpu as pltpu
from jax.experimental.pallas import tpu_sc as plsc

VOCAB = 1000000
EMBED_DIM = 64
BATCH = 16384
HIST = 50

NC = 2
NS = 16
NW = NC * NS

CB = 128
BPW = BATCH // NW
KPW = BPW // CB
UNITS = HIST * KPW
NBUF = 4

_mesh = plsc.VectorSubcoreMesh(
    core_axis_name="c", subcore_axis_name="s", num_cores=NC, num_subcores=NS
)


@functools.partial(
    pl.kernel,
    out_type=jax.ShapeDtypeStruct((HIST, 8, BATCH // CB, 8, CB), jnp.float32),
    mesh=_mesh,
    scratch_types=[
        pltpu.VMEM((HIST, KPW, CB), jnp.int32),
        pltpu.VMEM((NBUF, CB, EMBED_DIM), jnp.float32),
        pltpu.VMEM((NBUF, 8, 8, CB), jnp.float32),
        [pltpu.SemaphoreType.DMA] * NBUF,
        [pltpu.SemaphoreType.DMA] * NBUF,
    ],
    compiler_params=pltpu.CompilerParams(
        use_tc_tiling_on_sc=False, needs_layout_passes=False
    ),
)
def _gather_kernel(xt_hbm, table_hbm, out_hbm, idx_v, g_v, t_v, gsems, wsems):
    wid = lax.axis_index("s") * NC + lax.axis_index("c")
    pltpu.sync_copy(xt_hbm.at[:, pl.ds(wid * KPW, KPW), :], idx_v)

    def unit_hk(u):
        return u // KPW, lax.rem(u, KPW)

    def fire_gather(u, slot):
        h, k = unit_hk(u)
        pltpu.async_copy(
            table_hbm.at[idx_v.at[h, k]], g_v.at[slot], gsems[slot]
        )

    def out_ref_for(u):
        h, k = unit_hk(u)
        return out_hbm.at[h, :, wid * KPW + k]

    lane = lax.broadcasted_iota(jnp.int32, (16,), 0)

    def do_unit(u, slot):
        h, k = unit_hk(u)
        pltpu.make_async_copy(
            table_hbm.at[idx_v.at[h, k]], g_v.at[slot], gsems[slot]
        ).wait()

        @pl.when(u >= NBUF)
        def _():
            pltpu.make_async_copy(
                t_v.at[slot], out_ref_for(u - NBUF), wsems[slot]
            ).wait()

        @plsc.parallel_loop(0, 8, unroll=4)
        def _(td):
            dbase = jnp.full((16,), td * 8, jnp.int32)
            for dr in range(8):
                dcol = dbase + dr
                for c in range(CB // 16):
                    v = plsc.load_gather(
                        g_v.at[slot], [lane + (16 * c), dcol]
                    )
                    t_v[slot, td, dr, pl.ds(16 * c, 16)] = v

        pltpu.async_copy(t_v.at[slot], out_ref_for(u), wsems[slot])

        @pl.when(u + NBUF < UNITS)
        def _():
            fire_gather(u + NBUF, slot)

    for b in range(NBUF):
        fire_gather(b, b)

    def quad(p, carry):
        for b in range(NBUF):
            do_unit(NBUF * p + b, b)
        return carry

    lax.fori_loop(0, UNITS // NBUF, quad, 0)

    for b in range(NBUF):
        u = UNITS - NBUF + b
        pltpu.make_async_copy(t_v.at[b], out_ref_for(u), wsems[b]).wait()


def kernel(X, table):
    xt = X.T.reshape(HIST, BATCH // CB, CB).astype(jnp.int32)
    p = _gather_kernel(xt, table)
    return p.transpose(2, 4, 0, 1, 3).reshape(BATCH, HIST, EMBED_DIM)


# speedup vs baseline: 1.0288x; 1.0288x over previous
import functools
import jax
import jax.numpy as jnp
from jax import lax
from jax.experimental import pallas as pl
from jax.experimental.pallas import tpu as pltpu
from jax.experimental.pallas import tpu_sc as plsc

VOCAB = 1000000
EMBED_DIM = 64
BATCH = 16384
HIST = 50

NC = 2
NS = 16
NW = NC * NS

B = BATCH * HIST
B_PER_W = B // NW
CH = 256
CHUNKS = B_PER_W // CH
NBUF = 5
LOOK = 3
NITER = CHUNKS // NBUF

_mesh = plsc.VectorSubcoreMesh(
    core_axis_name="c", subcore_axis_name="s", num_cores=NC, num_subcores=NS
)


@functools.partial(
    pl.kernel,
    out_type=jax.ShapeDtypeStruct((B, EMBED_DIM), jnp.float32),
    mesh=_mesh,
    scratch_types=[
        pltpu.VMEM((CHUNKS, CH), jnp.int32),
        pltpu.VMEM((NBUF, CH, EMBED_DIM), jnp.float32),
        [pltpu.SemaphoreType.DMA] * NBUF,
        [pltpu.SemaphoreType.DMA] * NBUF,
    ],
    compiler_params=pltpu.CompilerParams(use_tc_tiling_on_sc=False),
)
def _gather_kernel(idx_hbm, table_hbm, out_hbm, idx_v, rows_v, gsems, osems):
    wid = lax.axis_index("s") * NC + lax.axis_index("c")
    base = wid * B_PER_W
    pltpu.sync_copy(idx_hbm.at[wid], idx_v)

    def fire_gather(chunk, slot):
        pltpu.async_copy(table_hbm.at[idx_v.at[chunk]], rows_v.at[slot], gsems[slot])

    def out_ref_for(chunk):
        return out_hbm.at[pl.ds(base + chunk * CH, CH)]

    for b in range(LOOK):
        fire_gather(b, b)

    def titer(t, carry):
        for b in range(NBUF):
            i = t * NBUF + b
            j = i + LOOK
            sj = (b + LOOK) % NBUF

            @pl.when(j < CHUNKS)
            def _():
                @pl.when(j >= NBUF)
                def _():
                    pltpu.make_async_copy(
                        rows_v.at[sj], out_ref_for(j - NBUF), osems[sj]
                    ).wait()
                fire_gather(j, sj)

            pltpu.make_async_copy(
                table_hbm.at[idx_v.at[i]], rows_v.at[b], gsems[b]
            ).wait()
            pltpu.async_copy(rows_v.at[b], out_ref_for(i), osems[b])
        return carry

    lax.fori_loop(0, NITER, titer, 0)

    for b in range(NBUF):
        last = CHUNKS - NBUF + b
        pltpu.make_async_copy(rows_v.at[b], out_ref_for(last), osems[b]).wait()


def kernel(X, table):
    idx = X.reshape(NW, CHUNKS, CH).astype(jnp.int32)
    out = _gather_kernel(idx, table)
    return out.reshape(BATCH, HIST, EMBED_DIM)
